# Initial kernel scaffold; baseline (speedup 1.0000x reference)
import functools

import jax
import jax.numpy as jnp
from jax import lax
from jax.experimental import pallas as pl
from jax.experimental.pallas import tpu as pltpu
from jax.experimental.pallas import tpu_sc as plsc

N = 10000
E = 320000
D_IN = 128
HID = 64
D_OUT = 64

NC = 2
NS = 16
NW = NC * NS
CH = 128
C = 80
E_PAD = NW * C * CH
NROWS = 10016
STRIPE = NROWS // NS
NBUF = 2
B_TC = 1000

_mesh = plsc.VectorSubcoreMesh(core_axis_name="c", subcore_axis_name="s")


@functools.partial(
    pl.kernel,
    out_type=jax.ShapeDtypeStruct((NC, NROWS, 16), jnp.float32),
    mesh=_mesh,
    scratch_types=[
        pltpu.VMEM((C, CH), jnp.int32),
        pltpu.VMEM((CH, 16), jnp.float32),
        pltpu.VMEM_SHARED((NROWS, 16), jnp.float32),
    ],
)
def _degree_kernel(dst_hbm, zrow_hbm, ones_hbm, out_hbm, dst_v, ones_v, hist_sh):
    cid = lax.axis_index("c")
    sid = lax.axis_index("s")
    wid = cid * NS + sid
    pltpu.sync_copy(zrow_hbm, hist_sh.at[pl.ds(sid * STRIPE, STRIPE)])
    pltpu.sync_copy(dst_hbm.at[wid], dst_v)
    pltpu.sync_copy(ones_hbm, ones_v)
    plsc.subcore_barrier()

    @pl.loop(0, C)
    def _edge_chunk(j):
        pltpu.sync_copy(ones_v, hist_sh.at[dst_v.at[j]], add=True)

    plsc.subcore_barrier()
    pltpu.sync_copy(
        hist_sh.at[pl.ds(sid * STRIPE, STRIPE)],
        out_hbm.at[cid].at[pl.ds(sid * STRIPE, STRIPE)],
    )


@functools.partial(
    pl.kernel,
    out_type=jax.ShapeDtypeStruct((NC, NROWS, HID), jnp.float32),
    mesh=_mesh,
    scratch_types=[
        pltpu.VMEM((C, CH), jnp.int32),
        pltpu.VMEM((C, CH), jnp.int32),
        pltpu.VMEM((NBUF, CH, HID), jnp.float32),
        pltpu.VMEM_SHARED((NROWS, HID), jnp.float32),
        pltpu.SemaphoreType.DMA,
        pltpu.SemaphoreType.DMA,
    ],
)
def _scatter_kernel(g_hbm, src_hbm, dst_hbm, zblk_hbm, out_hbm,
                    src_v, dst_v, rows_v, agg_sh, sem0, sem1):
    cid = lax.axis_index("c")
    sid = lax.axis_index("s")
    wid = cid * NS + sid
    sems = (sem0, sem1)
    pltpu.sync_copy(zblk_hbm, agg_sh.at[pl.ds(sid * STRIPE, STRIPE)])
    pltpu.sync_copy(src_hbm.at[wid], src_v)
    pltpu.sync_copy(dst_hbm.at[wid], dst_v)
    plsc.subcore_barrier()

    for b in range(NBUF):
        pltpu.async_copy(g_hbm.at[src_v.at[b]], rows_v.at[b], sems[b])

    @pl.loop(0, C, step=NBUF)
    def _chunk(j):
        for b in range(NBUF):
            pltpu.make_async_copy(
                g_hbm.at[src_v.at[j + b]], rows_v.at[b], sems[b]
            ).wait()
            pltpu.sync_copy(rows_v.at[b], agg_sh.at[dst_v.at[j + b]], add=True)

            @pl.when(j + b + NBUF < C)
            def _start_next():
                pltpu.async_copy(
                    g_hbm.at[src_v.at[j + b + NBUF]], rows_v.at[b], sems[b]
                )

    plsc.subcore_barrier()
    pltpu.sync_copy(
        agg_sh.at[pl.ds(sid * STRIPE, STRIPE)],
        out_hbm.at[cid].at[pl.ds(sid * STRIPE, STRIPE)],
    )


def _matmul_scale(x, W1, h0, h1):
    def body(x_ref, w_ref, h0_ref, h1_ref, g_ref):
        deg = h0_ref[:, 0:1] + h1_ref[:, 0:1] + 1.0
        dis = lax.rsqrt(deg)
        h = lax.dot_general(
            x_ref[...], w_ref[...], (((1,), (0,)), ((), ())),
            precision=lax.Precision.HIGHEST,
            preferred_element_type=jnp.float32,
        )
        g_ref[...] = h * dis

    return pl.pallas_call(
        body,
        grid=(N // B_TC,),
        in_specs=[
            pl.BlockSpec((B_TC, D_IN), lambda i: (i, 0)),
            pl.BlockSpec((D_IN, HID), lambda i: (0, 0)),
            pl.BlockSpec((B_TC, 16), lambda i: (i, 0)),
            pl.BlockSpec((B_TC, 16), lambda i: (i, 0)),
        ],
        out_specs=pl.BlockSpec((B_TC, HID), lambda i: (i, 0)),
        out_shape=jax.ShapeDtypeStruct((N, HID), jnp.float32),
    )(x, W1, h0, h1)


def _finish(agg0, agg1, g, h0, h1, b1, W2, b2):
    def body(a0_ref, a1_ref, g_ref, h0_ref, h1_ref, b1_ref, w2_ref, b2_ref,
             out_ref, emb_ref):
        deg = h0_ref[:, 0:1] + h1_ref[:, 0:1] + 1.0
        dis = lax.rsqrt(deg)
        s = (a0_ref[...] + a1_ref[...] + g_ref[...]) * dis + b1_ref[...]
        emb = jnp.maximum(s, 0.0)
        emb_ref[...] = emb
        out_ref[...] = lax.dot_general(
            emb, w2_ref[...], (((1,), (0,)), ((), ())),
            precision=lax.Precision.HIGHEST,
            preferred_element_type=jnp.float32,
        ) + b2_ref[...]

    return pl.pallas_call(
        body,
        grid=(N // B_TC,),
        in_specs=[
            pl.BlockSpec((B_TC, HID), lambda i: (i, 0)),
            pl.BlockSpec((B_TC, HID), lambda i: (i, 0)),
            pl.BlockSpec((B_TC, HID), lambda i: (i, 0)),
            pl.BlockSpec((B_TC, 16), lambda i: (i, 0)),
            pl.BlockSpec((B_TC, 16), lambda i: (i, 0)),
            pl.BlockSpec((1, HID), lambda i: (0, 0)),
            pl.BlockSpec((HID, D_OUT), lambda i: (0, 0)),
            pl.BlockSpec((1, D_OUT), lambda i: (0, 0)),
        ],
        out_specs=[
            pl.BlockSpec((B_TC, D_OUT), lambda i: (i, 0)),
            pl.BlockSpec((B_TC, HID), lambda i: (i, 0)),
        ],
        out_shape=[
            jax.ShapeDtypeStruct((N, D_OUT), jnp.float32),
            jax.ShapeDtypeStruct((N, HID), jnp.float32),
        ],
    )(agg0, agg1, g, h0, h1, b1, W2, b2)


def kernel(x, edge_index, W1, b1, W2, b2):
    src = edge_index[0]
    dst = edge_index[1]
    pad = E_PAD - E
    src_p = jnp.concatenate([src, jnp.zeros((pad,), jnp.int32)]).reshape(NW, C, CH)
    dst_p = jnp.concatenate([dst, jnp.full((pad,), N, jnp.int32)]).reshape(NW, C, CH)
    zrow = jnp.zeros((STRIPE, 16), jnp.float32)
    ones_rows = jnp.zeros((CH, 16), jnp.float32).at[:, 0].set(1.0)
    zblk = jnp.zeros((STRIPE, HID), jnp.float32)

    hist = _degree_kernel(dst_p, zrow, ones_rows)
    h0, h1 = hist[0], hist[1]
    g = _matmul_scale(x, W1, h0, h1)
    agg = _scatter_kernel(g, src_p, dst_p, zblk)
    out, emb = _finish(agg[0], agg[1], g, h0, h1,
                       b1.reshape(1, HID), W2, b2.reshape(1, D_OUT))
    return out, emb


# speedup vs baseline: 20.9668x; 20.9668x over previous
import functools

import jax
import jax.numpy as jnp
from jax import lax
from jax.experimental import pallas as pl
from jax.experimental.pallas import tpu as pltpu
from jax.experimental.pallas import tpu_sc as plsc

N = 10000
E = 320000
D_IN = 128
HID = 64
D_OUT = 64

NC = 2
NS = 16
NW = NC * NS
CH = 128
C = 80
E_PAD = NW * C * CH
NROWS = 10112
STRIPE = NROWS // NS
NBUF = 2
B_TC = 1000

_mesh = plsc.VectorSubcoreMesh(core_axis_name="c", subcore_axis_name="s")


@functools.partial(
    pl.kernel,
    out_type=jax.ShapeDtypeStruct((NC, NROWS, 16), jnp.float32),
    mesh=_mesh,
    scratch_types=[
        pltpu.VMEM((C, CH), jnp.int32),
        pltpu.VMEM((CH, 16), jnp.float32),
        pltpu.VMEM_SHARED((NROWS, 16), jnp.float32),
    ],
    compiler_params=pltpu.CompilerParams(use_tc_tiling_on_sc=False),
)
def _degree_kernel(dst_hbm, zrow_hbm, ones_hbm, out_hbm, dst_v, ones_v, hist_sh):
    cid = lax.axis_index("c")
    sid = lax.axis_index("s")
    wid = cid * NS + sid
    pltpu.sync_copy(zrow_hbm, hist_sh.at[pl.ds(sid * STRIPE, STRIPE)])
    pltpu.sync_copy(dst_hbm.at[wid], dst_v)
    pltpu.sync_copy(ones_hbm, ones_v)
    plsc.subcore_barrier()

    @pl.loop(0, C)
    def _edge_chunk(j):
        pltpu.sync_copy(ones_v, hist_sh.at[dst_v.at[j]], add=True)

    plsc.subcore_barrier()
    pltpu.sync_copy(
        hist_sh.at[pl.ds(sid * STRIPE, STRIPE)],
        out_hbm.at[cid].at[pl.ds(sid * STRIPE, STRIPE)],
    )


@functools.partial(
    pl.kernel,
    out_type=jax.ShapeDtypeStruct((NC, NROWS, HID), jnp.float32),
    mesh=_mesh,
    scratch_types=[
        pltpu.VMEM((C, CH), jnp.int32),
        pltpu.VMEM((C, CH), jnp.int32),
        pltpu.VMEM((NBUF, CH, HID), jnp.float32),
        pltpu.VMEM_SHARED((NROWS, HID), jnp.float32),
        pltpu.SemaphoreType.DMA,
        pltpu.SemaphoreType.DMA,
    ],
    compiler_params=pltpu.CompilerParams(use_tc_tiling_on_sc=False),
)
def _scatter_kernel(g_hbm, src_hbm, dst_hbm, zblk_hbm, out_hbm,
                    src_v, dst_v, rows_v, agg_sh, sem0, sem1):
    cid = lax.axis_index("c")
    sid = lax.axis_index("s")
    wid = cid * NS + sid
    sems = (sem0, sem1)
    pltpu.sync_copy(zblk_hbm, agg_sh.at[pl.ds(sid * STRIPE, STRIPE)])
    pltpu.sync_copy(src_hbm.at[wid], src_v)
    pltpu.sync_copy(dst_hbm.at[wid], dst_v)
    plsc.subcore_barrier()

    for b in range(NBUF):
        pltpu.async_copy(g_hbm.at[src_v.at[b]], rows_v.at[b], sems[b])

    @pl.loop(0, C, step=NBUF)
    def _chunk(j):
        for b in range(NBUF):
            pltpu.make_async_copy(
                g_hbm.at[src_v.at[j + b]], rows_v.at[b], sems[b]
            ).wait()
            pltpu.sync_copy(rows_v.at[b], agg_sh.at[dst_v.at[j + b]], add=True)

            @pl.when(j + b + NBUF < C)
            def _start_next():
                pltpu.async_copy(
                    g_hbm.at[src_v.at[j + b + NBUF]], rows_v.at[b], sems[b]
                )

    plsc.subcore_barrier()
    pltpu.sync_copy(
        agg_sh.at[pl.ds(sid * STRIPE, STRIPE)],
        out_hbm.at[cid].at[pl.ds(sid * STRIPE, STRIPE)],
    )


def _matmul_scale(x, W1, h0, h1):
    def body(x_ref, w_ref, h0_ref, h1_ref, g_ref):
        deg = h0_ref[:, 0:1] + h1_ref[:, 0:1] + 1.0
        dis = lax.rsqrt(deg)
        h = lax.dot_general(
            x_ref[...], w_ref[...], (((1,), (0,)), ((), ())),
            precision=lax.Precision.HIGHEST,
            preferred_element_type=jnp.float32,
        )
        g_ref[...] = h * dis

    return pl.pallas_call(
        body,
        grid=(N // B_TC,),
        in_specs=[
            pl.BlockSpec((B_TC, D_IN), lambda i: (i, 0)),
            pl.BlockSpec((D_IN, HID), lambda i: (0, 0)),
            pl.BlockSpec((B_TC, 16), lambda i: (i, 0)),
            pl.BlockSpec((B_TC, 16), lambda i: (i, 0)),
        ],
        out_specs=pl.BlockSpec((B_TC, HID), lambda i: (i, 0)),
        out_shape=jax.ShapeDtypeStruct((N, HID), jnp.float32),
    )(x, W1, h0, h1)


def _finish(agg0, agg1, g, h0, h1, b1, W2, b2):
    def body(a0_ref, a1_ref, g_ref, h0_ref, h1_ref, b1_ref, w2_ref, b2_ref,
             out_ref, emb_ref):
        deg = h0_ref[:, 0:1] + h1_ref[:, 0:1] + 1.0
        dis = lax.rsqrt(deg)
        s = (a0_ref[...] + a1_ref[...] + g_ref[...]) * dis + b1_ref[...]
        emb = jnp.maximum(s, 0.0)
        emb_ref[...] = emb
        out_ref[...] = lax.dot_general(
            emb, w2_ref[...], (((1,), (0,)), ((), ())),
            precision=lax.Precision.HIGHEST,
            preferred_element_type=jnp.float32,
        ) + b2_ref[...]

    return pl.pallas_call(
        body,
        grid=(N // B_TC,),
        in_specs=[
            pl.BlockSpec((B_TC, HID), lambda i: (i, 0)),
            pl.BlockSpec((B_TC, HID), lambda i: (i, 0)),
            pl.BlockSpec((B_TC, HID), lambda i: (i, 0)),
            pl.BlockSpec((B_TC, 16), lambda i: (i, 0)),
            pl.BlockSpec((B_TC, 16), lambda i: (i, 0)),
            pl.BlockSpec((1, HID), lambda i: (0, 0)),
            pl.BlockSpec((HID, D_OUT), lambda i: (0, 0)),
            pl.BlockSpec((1, D_OUT), lambda i: (0, 0)),
        ],
        out_specs=[
            pl.BlockSpec((B_TC, D_OUT), lambda i: (i, 0)),
            pl.BlockSpec((B_TC, HID), lambda i: (i, 0)),
        ],
        out_shape=[
            jax.ShapeDtypeStruct((N, D_OUT), jnp.float32),
            jax.ShapeDtypeStruct((N, HID), jnp.float32),
        ],
    )(agg0, agg1, g, h0, h1, b1, W2, b2)


def kernel(x, edge_index, W1, b1, W2, b2):
    src = edge_index[0]
    dst = edge_index[1]
    pad = E_PAD - E
    src_p = jnp.concatenate([src, jnp.zeros((pad,), jnp.int32)]).reshape(NW, C, CH)
    dst_p = jnp.concatenate([dst, jnp.full((pad,), N, jnp.int32)]).reshape(NW, C, CH)
    zrow = jnp.zeros((STRIPE, 16), jnp.float32)
    ones_rows = jnp.zeros((CH, 16), jnp.float32).at[:, 0].set(1.0)
    zblk = jnp.zeros((STRIPE, HID), jnp.float32)

    hist = _degree_kernel(dst_p, zrow, ones_rows)
    h0, h1 = hist[0], hist[1]
    g = _matmul_scale(x, W1, h0, h1)
    agg = _scatter_kernel(g, src_p, dst_p, zblk)
    out, emb = _finish(agg[0], agg[1], g, h0, h1,
                       b1.reshape(1, HID), W2, b2.reshape(1, D_OUT))
    return out, emb
